# baseline (speedup 1.0000x reference)
import functools

import jax
import jax.numpy as jnp
from jax.experimental import pallas as pl
from jax.experimental.pallas import tpu as pltpu

CHANNELS = 768
N_CLUSTERS = 512
N_TOKENS = 32768
BT = 2048
HB = 1024
N_BLOCKS = N_TOKENS // BT


def _cluster_body(x_ref, w_ref, b_ref, out_ref, cnt_ref):
    i = pl.program_id(0)

    @pl.when(i == 0)
    def _init():
        out_ref[...] = jnp.zeros_like(out_ref)
        cnt_ref[...] = jnp.zeros_like(cnt_ref)

    w = w_ref[...]
    bias = b_ref[...]
    sums = []
    cnts = []
    for h in range(BT // HB):
        xb = x_ref[pl.ds(h * HB, HB), :]
        logits = (
            jnp.dot(xb, w.T, preferred_element_type=jnp.float32) + bias
        )
        rowmax = jnp.max(logits, axis=1, keepdims=True)
        onehot = (logits == rowmax).astype(jnp.bfloat16)
        sums.append(
            jax.lax.dot_general(
                onehot,
                xb.astype(jnp.bfloat16),
                (((0,), (0,)), ((), ())),
                preferred_element_type=jnp.float32,
            )
        )
        cnts.append(jnp.sum(onehot.astype(jnp.float32), axis=0, keepdims=True))
    out_ref[...] += sum(sums)
    cnt_ref[...] += sum(cnts)

    @pl.when(i == N_BLOCKS - 1)
    def _finalize():
        out_ref[...] = out_ref[...] / cnt_ref[...].T


@jax.jit
def kernel(x, W, b):
    out = pl.pallas_call(
        _cluster_body,
        grid=(N_BLOCKS,),
        in_specs=[
            pl.BlockSpec((BT, CHANNELS), lambda i: (i, 0)),
            pl.BlockSpec((N_CLUSTERS, CHANNELS), lambda i: (0, 0)),
            pl.BlockSpec((1, N_CLUSTERS), lambda i: (0, 0)),
        ],
        out_specs=pl.BlockSpec((N_CLUSTERS, CHANNELS), lambda i: (0, 0)),
        out_shape=jax.ShapeDtypeStruct((N_CLUSTERS, CHANNELS), jnp.float32),
        scratch_shapes=[pltpu.VMEM((1, N_CLUSTERS), jnp.float32)],
    )(x, W, b.reshape(1, N_CLUSTERS))
    return out


# speedup vs baseline: 1.0465x; 1.0465x over previous
import functools

import jax
import jax.numpy as jnp
from jax.experimental import pallas as pl
from jax.experimental.pallas import tpu as pltpu

CHANNELS = 768
N_CLUSTERS = 512
N_TOKENS = 32768
BT = 4096
HB = 2048
N_BLOCKS = N_TOKENS // BT


def _cluster_body(x_ref, w_ref, b_ref, out_ref, cnt_ref):
    i = pl.program_id(0)

    @pl.when(i == 0)
    def _init():
        out_ref[...] = jnp.zeros_like(out_ref)
        cnt_ref[...] = jnp.zeros_like(cnt_ref)

    w = w_ref[...]
    bias = b_ref[...]
    sums = []
    cnts = []
    for h in range(BT // HB):
        xb = x_ref[pl.ds(h * HB, HB), :]
        logits = (
            jnp.dot(xb, w.T, preferred_element_type=jnp.float32) + bias
        )
        rowmax = jnp.max(logits, axis=1, keepdims=True)
        onehot = (logits == rowmax).astype(jnp.bfloat16)
        sums.append(
            jax.lax.dot_general(
                onehot,
                xb.astype(jnp.bfloat16),
                (((0,), (0,)), ((), ())),
                preferred_element_type=jnp.float32,
            )
        )
        cnts.append(jnp.sum(onehot.astype(jnp.float32), axis=0, keepdims=True))
    out_ref[...] += sum(sums)
    cnt_ref[...] += sum(cnts)

    @pl.when(i == N_BLOCKS - 1)
    def _finalize():
        out_ref[...] = out_ref[...] / cnt_ref[...].T


@jax.jit
def kernel(x, W, b):
    out = pl.pallas_call(
        _cluster_body,
        grid=(N_BLOCKS,),
        in_specs=[
            pl.BlockSpec((BT, CHANNELS), lambda i: (i, 0)),
            pl.BlockSpec((N_CLUSTERS, CHANNELS), lambda i: (0, 0)),
            pl.BlockSpec((1, N_CLUSTERS), lambda i: (0, 0)),
        ],
        out_specs=pl.BlockSpec((N_CLUSTERS, CHANNELS), lambda i: (0, 0)),
        out_shape=jax.ShapeDtypeStruct((N_CLUSTERS, CHANNELS), jnp.float32),
        scratch_shapes=[pltpu.VMEM((1, N_CLUSTERS), jnp.float32)],
    )(x, W, b.reshape(1, N_CLUSTERS))
    return out
